# baseline (speedup 1.0000x reference)
import dataclasses

import jax
import jax.numpy as jnp
from jax import lax
from jax.experimental import pallas as pl
from jax.experimental.pallas import tpu as pltpu
from jax.experimental.pallas import tpu_sc as plsc

_TP = 4096
_NC = 2
_NS = 16


def _prep_body(f_ref, i_ref, ft_ref, it_ref):
    fr = f_ref[0]
    z = jnp.zeros((fr.shape[0], 5), jnp.float32)
    ft_ref[0] = jnp.concatenate([fr, z], axis=1).T[:3]
    it_ref[0] = i_ref[0].T


def _prep(f, knn_idx):
    B, N, C = f.shape
    K = knn_idx.shape[2]
    return pl.pallas_call(
        _prep_body,
        grid=(B,),
        in_specs=[
            pl.BlockSpec((1, N, C), lambda b: (b, 0, 0)),
            pl.BlockSpec((1, N, K), lambda b: (b, 0, 0)),
        ],
        out_specs=[
            pl.BlockSpec((1, C, N), lambda b: (b, 0, 0)),
            pl.BlockSpec((1, K, N), lambda b: (b, 0, 0)),
        ],
        out_shape=[
            jax.ShapeDtypeStruct((B, C, N), jnp.float32),
            jax.ShapeDtypeStruct((B, K, N), jnp.int32),
        ],
    )(f, knn_idx)


def _sc_gather(fT, idxT):
    B = fT.shape[0]
    K, N = idxT.shape[1], idxT.shape[2]
    C = fT.shape[1] // N
    nw = _NC * _NS
    ppw = N // nw
    mesh = plsc.VectorSubcoreMesh(core_axis_name="c", subcore_axis_name="s")
    cp = pltpu.CompilerParams()
    if "needs_layout_passes" in pltpu.CompilerParams.__dataclass_fields__:
        cp = dataclasses.replace(cp, needs_layout_passes=False)

    @pl.kernel(
        out_type=jax.ShapeDtypeStruct((B, K * C, N), jnp.float32),
        mesh=mesh,
        compiler_params=cp,
        scratch_types=[
            pltpu.VMEM((C * N,), jnp.float32),
            pltpu.VMEM((C * N,), jnp.float32),
            pltpu.VMEM((K, ppw), jnp.int32),
            pltpu.VMEM((K, ppw), jnp.int32),
            pltpu.VMEM((K * C, ppw), jnp.float32),
            pltpu.SemaphoreType.DMA((2,)),
            pltpu.SemaphoreType.DMA((2,)),
            pltpu.SemaphoreType.DMA((2,)),
        ],
    )
    def gather_kernel(f_hbm, i_hbm, o_hbm, tab0, tab1, idx0, idx1,
                      out0, tsem, isem, osem):
        tab_v = [tab0, tab1]
        idx_v = [idx0, idx1]
        wid = lax.axis_index("s") * _NC + lax.axis_index("c")
        base = wid * ppw

        def start_in(b, s):
            t = pltpu.async_copy(f_hbm.at[b], tab_v[s], tsem.at[s])
            i = pltpu.async_copy(i_hbm.at[b, :, pl.ds(base, ppw)],
                                 idx_v[s], isem.at[s])
            return t, i

        in_cp = {0: start_in(0, 0)}
        out_cp = {}
        for b in range(B):
            s = b % 2
            if b + 1 < B:
                in_cp[b + 1] = start_in(b + 1, 1 - s)
            t, i = in_cp.pop(b)
            t.wait()
            i.wait()
            if b >= 1:
                out_cp.pop(b - 1).wait()

            @pl.loop(0, ppw // 16)
            def _(g):
                for k in range(K):
                    knn = idx_v[s][k, pl.ds(g * 16, 16)]
                    for c in range(C):
                        addr = knn + (c * N) if c else knn
                        vals = plsc.load_gather(tab_v[s], [addr])
                        out0[k * C + c, pl.ds(g * 16, 16)] = vals

            out_cp[b] = pltpu.async_copy(
                out0, o_hbm.at[b, :, pl.ds(base, ppw)], osem.at[b % 2])
        out_cp.pop(B - 1).wait()

    return gather_kernel(fT, idxT)


def _dense_body(g_ref, f3_ref, wcat_ref, b1_ref, w2b_ref, b2_ref,
                w3_ref, b3_ref, w4_ref, b4_ref, w5_ref, b5_ref, out_ref):
    g = g_ref[0]
    f3 = f3_ref[0]
    xin = jnp.concatenate([g, f3], axis=0)
    h = jnp.dot(wcat_ref[...], xin.astype(jnp.bfloat16),
                preferred_element_type=jnp.float32)
    h = jnp.maximum(h + b1_ref[...], 0.0)
    h = jnp.dot(w2b_ref[...], h.astype(jnp.bfloat16),
                preferred_element_type=jnp.float32)
    h = jnp.maximum(h + b2_ref[...], 0.0)
    m = jnp.maximum(h[:256], h[256:])
    m = jnp.maximum(m[:128], m[128:])
    m = jnp.maximum(m[:64], m[64:])
    xmax = jnp.maximum(m[:32], m[32:])
    t = jnp.dot(w3_ref[...], f3, preferred_element_type=jnp.float32)
    t = jnp.maximum(t + b3_ref[...], 0.0)
    gsk = jnp.dot(w4_ref[...], t, preferred_element_type=jnp.float32)
    gsk = jnp.maximum(gsk + b4_ref[...], 0.0)
    s = xmax + gsk
    o8 = (jnp.dot(w5_ref[...], s, preferred_element_type=jnp.float32)
          + b5_ref[...])
    out_ref[0] = o8.T[:, :3]


def _blkdiag(w, r):
    a, b = w.shape
    out = jnp.zeros((r * a, r * b), w.dtype)
    for i in range(r):
        out = out.at[i * a:(i + 1) * a, i * b:(i + 1) * b].set(w)
    return out


def kernel(f, knn_idx, W1, b1, W2, b2, W3, b3, W4, b4, W5, b5):
    B, N, C = f.shape
    K = knn_idx.shape[2]

    fT, idxT = _prep(f, knn_idx)
    g3p = _sc_gather(fT.reshape(B, C * N), idxT)

    W1a = W1[:C] + W1[C:]
    W1b = W1[C:]
    wcat = jnp.concatenate([_blkdiag(W1a, K),
                            jnp.tile(-W1b, (1, K))],
                           axis=0).T.astype(jnp.bfloat16)
    b1r = jnp.tile(b1, K)[:, None]
    w2b = _blkdiag(W2, K).T.astype(jnp.bfloat16)
    b2r = jnp.tile(b2, K)[:, None]
    w3t = W3.T
    w4t = W4.T
    w5t = jnp.pad(W5.T, ((0, 5), (0, 0)))
    b5p = jnp.pad(b5, (0, 5))[:, None]

    grid = (B, N // _TP)
    out = pl.pallas_call(
        _dense_body,
        grid=grid,
        in_specs=[
            pl.BlockSpec((1, K * C, _TP), lambda b, i: (b, 0, i)),
            pl.BlockSpec((1, C, _TP), lambda b, i: (b, 0, i)),
            pl.BlockSpec((K * 32, K * C + 3), lambda b, i: (0, 0)),
            pl.BlockSpec((K * 32, 1), lambda b, i: (0, 0)),
            pl.BlockSpec((K * 32, K * 32), lambda b, i: (0, 0)),
            pl.BlockSpec((K * 32, 1), lambda b, i: (0, 0)),
            pl.BlockSpec((32, 3), lambda b, i: (0, 0)),
            pl.BlockSpec((32, 1), lambda b, i: (0, 0)),
            pl.BlockSpec((32, 32), lambda b, i: (0, 0)),
            pl.BlockSpec((32, 1), lambda b, i: (0, 0)),
            pl.BlockSpec((8, 32), lambda b, i: (0, 0)),
            pl.BlockSpec((8, 1), lambda b, i: (0, 0)),
        ],
        out_specs=pl.BlockSpec((1, _TP, 3), lambda b, i: (b, i, 0)),
        out_shape=jax.ShapeDtypeStruct((B, N, 3), jnp.float32),
    )(g3p, fT, wcat, b1r, w2b, b2r, w3t, b3[:, None], w4t, b4[:, None],
      w5t, b5p)
    return out


# speedup vs baseline: 1.3465x; 1.3465x over previous
import dataclasses

import jax
import jax.numpy as jnp
from jax import lax
from jax.experimental import pallas as pl
from jax.experimental.pallas import tpu as pltpu
from jax.experimental.pallas import tpu_sc as plsc

_TP = 4096
_NC = 2
_NS = 16


def _prep_body(f_ref, i_ref, ft_ref, it_ref):
    fr = f_ref[0]
    z = jnp.zeros((fr.shape[0], 5), jnp.float32)
    ft_ref[0] = jnp.concatenate([fr, z], axis=1).T[:3]
    it_ref[0] = i_ref[0].T


def _prep(f, knn_idx):
    B, N, C = f.shape
    K = knn_idx.shape[2]
    return pl.pallas_call(
        _prep_body,
        grid=(B,),
        in_specs=[
            pl.BlockSpec((1, N, C), lambda b: (b, 0, 0)),
            pl.BlockSpec((1, N, K), lambda b: (b, 0, 0)),
        ],
        out_specs=[
            pl.BlockSpec((1, C, N), lambda b: (b, 0, 0)),
            pl.BlockSpec((1, K, N), lambda b: (b, 0, 0)),
        ],
        out_shape=[
            jax.ShapeDtypeStruct((B, C, N), jnp.float32),
            jax.ShapeDtypeStruct((B, K, N), jnp.int32),
        ],
    )(f, knn_idx)


def _sc_gather(fT, idxT):
    B = fT.shape[0]
    K, N = idxT.shape[1], idxT.shape[2]
    C = fT.shape[1] // N
    nw = _NC * _NS
    ppw = N // nw
    mesh = plsc.VectorSubcoreMesh(core_axis_name="c", subcore_axis_name="s")
    cp = pltpu.CompilerParams()
    if "needs_layout_passes" in pltpu.CompilerParams.__dataclass_fields__:
        cp = dataclasses.replace(cp, needs_layout_passes=False)

    @pl.kernel(
        out_type=jax.ShapeDtypeStruct((B, K * C, N), jnp.float32),
        mesh=mesh,
        compiler_params=cp,
        scratch_types=[
            pltpu.VMEM((C * N,), jnp.float32),
            pltpu.VMEM((C * N,), jnp.float32),
            pltpu.VMEM((K, ppw), jnp.int32),
            pltpu.VMEM((K, ppw), jnp.int32),
            pltpu.VMEM((K * C, ppw), jnp.float32),
            pltpu.SemaphoreType.DMA((2,)),
            pltpu.SemaphoreType.DMA((2,)),
            pltpu.SemaphoreType.DMA((2,)),
        ],
    )
    def gather_kernel(f_hbm, i_hbm, o_hbm, tab0, tab1, idx0, idx1,
                      out0, tsem, isem, osem):
        tab_v = [tab0, tab1]
        idx_v = [idx0, idx1]
        wid = lax.axis_index("s") * _NC + lax.axis_index("c")
        base = wid * ppw

        def start_in(b, s):
            t = pltpu.async_copy(f_hbm.at[b], tab_v[s], tsem.at[s])
            i = pltpu.async_copy(i_hbm.at[b, :, pl.ds(base, ppw)],
                                 idx_v[s], isem.at[s])
            return t, i

        in_cp = {0: start_in(0, 0)}
        out_cp = {}
        for b in range(B):
            s = b % 2
            if b + 1 < B:
                in_cp[b + 1] = start_in(b + 1, 1 - s)
            t, i = in_cp.pop(b)
            t.wait()
            i.wait()
            if b >= 1:
                out_cp.pop(b - 1).wait()

            @pl.loop(0, ppw // 16)
            def _(g):
                for k in range(K):
                    knn = idx_v[s][k, pl.ds(g * 16, 16)]
                    for c in range(C):
                        addr = knn + (c * N) if c else knn
                        vals = plsc.load_gather(tab_v[s], [addr])
                        out0[k * C + c, pl.ds(g * 16, 16)] = vals

            out_cp[b] = pltpu.async_copy(
                out0, o_hbm.at[b, :, pl.ds(base, ppw)], osem.at[b % 2])
        out_cp.pop(B - 1).wait()

    return gather_kernel(fT, idxT)


def _dense_body(g_ref, f3_ref, wcat_ref, b1_ref, w2b_ref, b2_ref,
                w3_ref, b3_ref, w4_ref, b4_ref, w5_ref, b5_ref, out_ref):
    g = g_ref[0]
    f3 = f3_ref[0]
    xin = jnp.concatenate([g, f3], axis=0)
    h = jnp.dot(wcat_ref[...], xin.astype(jnp.bfloat16),
                preferred_element_type=jnp.float32)
    h = jnp.maximum(h + b1_ref[...], 0.0)
    h = jnp.dot(w2b_ref[...], h.astype(jnp.bfloat16),
                preferred_element_type=jnp.float32)
    h = jnp.maximum(h + b2_ref[...], 0.0)
    m = jnp.maximum(h[:256], h[256:])
    m = jnp.maximum(m[:128], m[128:])
    m = jnp.maximum(m[:64], m[64:])
    xmax = jnp.maximum(m[:32], m[32:])
    t = jnp.dot(w3_ref[...], f3, preferred_element_type=jnp.float32)
    t = jnp.maximum(t + b3_ref[...], 0.0)
    gsk = jnp.dot(w4_ref[...], t, preferred_element_type=jnp.float32)
    gsk = jnp.maximum(gsk + b4_ref[...], 0.0)
    s = xmax + gsk
    o8 = (jnp.dot(w5_ref[...], s, preferred_element_type=jnp.float32)
          + b5_ref[...])
    out_ref[0] = o8.T[:, :3]


def _blkdiag(w, r):
    a, b = w.shape
    out = jnp.zeros((r * a, r * b), w.dtype)
    for i in range(r):
        out = out.at[i * a:(i + 1) * a, i * b:(i + 1) * b].set(w)
    return out


def kernel(f, knn_idx, W1, b1, W2, b2, W3, b3, W4, b4, W5, b5):
    B, N, C = f.shape
    K = knn_idx.shape[2]

    fT = jnp.swapaxes(f, 1, 2)
    idxT = jnp.swapaxes(knn_idx, 1, 2)
    g3p = _sc_gather(fT.reshape(B, C * N), idxT)

    W1a = W1[:C] + W1[C:]
    W1b = W1[C:]
    wcat = jnp.concatenate([_blkdiag(W1a, K),
                            jnp.tile(-W1b, (1, K))],
                           axis=0).T.astype(jnp.bfloat16)
    b1r = jnp.tile(b1, K)[:, None]
    w2b = _blkdiag(W2, K).T.astype(jnp.bfloat16)
    b2r = jnp.tile(b2, K)[:, None]
    w3t = W3.T
    w4t = W4.T
    w5t = jnp.pad(W5.T, ((0, 5), (0, 0)))
    b5p = jnp.pad(b5, (0, 5))[:, None]

    grid = (B, N // _TP)
    out = pl.pallas_call(
        _dense_body,
        grid=grid,
        in_specs=[
            pl.BlockSpec((1, K * C, _TP), lambda b, i: (b, 0, i)),
            pl.BlockSpec((1, C, _TP), lambda b, i: (b, 0, i)),
            pl.BlockSpec((K * 32, K * C + 3), lambda b, i: (0, 0)),
            pl.BlockSpec((K * 32, 1), lambda b, i: (0, 0)),
            pl.BlockSpec((K * 32, K * 32), lambda b, i: (0, 0)),
            pl.BlockSpec((K * 32, 1), lambda b, i: (0, 0)),
            pl.BlockSpec((32, 3), lambda b, i: (0, 0)),
            pl.BlockSpec((32, 1), lambda b, i: (0, 0)),
            pl.BlockSpec((32, 32), lambda b, i: (0, 0)),
            pl.BlockSpec((32, 1), lambda b, i: (0, 0)),
            pl.BlockSpec((8, 32), lambda b, i: (0, 0)),
            pl.BlockSpec((8, 1), lambda b, i: (0, 0)),
        ],
        out_specs=pl.BlockSpec((1, _TP, 3), lambda b, i: (b, i, 0)),
        out_shape=jax.ShapeDtypeStruct((B, N, 3), jnp.float32),
    )(g3p, fT, wcat, b1r, w2b, b2r, w3t, b3[:, None], w4t, b4[:, None],
      w5t, b5p)
    return out


# speedup vs baseline: 1.5510x; 1.1519x over previous
import dataclasses

import jax
import jax.numpy as jnp
from jax import lax
from jax.experimental import pallas as pl
from jax.experimental.pallas import tpu as pltpu
from jax.experimental.pallas import tpu_sc as plsc

_TP = 4096
_NC = 2
_NS = 16


def _prep_body(f_ref, i_ref, ft_ref, it_ref):
    fr = f_ref[0]
    z = jnp.zeros((fr.shape[0], 5), jnp.float32)
    ft_ref[0] = jnp.concatenate([fr, z], axis=1).T[:3]
    it_ref[0] = i_ref[0].T


def _prep(f, knn_idx):
    B, N, C = f.shape
    K = knn_idx.shape[2]
    return pl.pallas_call(
        _prep_body,
        grid=(B,),
        in_specs=[
            pl.BlockSpec((1, N, C), lambda b: (b, 0, 0)),
            pl.BlockSpec((1, N, K), lambda b: (b, 0, 0)),
        ],
        out_specs=[
            pl.BlockSpec((1, C, N), lambda b: (b, 0, 0)),
            pl.BlockSpec((1, K, N), lambda b: (b, 0, 0)),
        ],
        out_shape=[
            jax.ShapeDtypeStruct((B, C, N), jnp.float32),
            jax.ShapeDtypeStruct((B, K, N), jnp.int32),
        ],
    )(f, knn_idx)


def _sc_gather(fT, idxT):
    B = fT.shape[0]
    K, N = idxT.shape[1], idxT.shape[2]
    C = fT.shape[1] // N
    nw = _NC * _NS
    ppw = N // nw
    mesh = plsc.VectorSubcoreMesh(core_axis_name="c", subcore_axis_name="s")
    cp = pltpu.CompilerParams()
    if "needs_layout_passes" in pltpu.CompilerParams.__dataclass_fields__:
        cp = dataclasses.replace(cp, needs_layout_passes=False)

    @pl.kernel(
        out_type=jax.ShapeDtypeStruct((B, K * C, N), jnp.float32),
        mesh=mesh,
        compiler_params=cp,
        scratch_types=[
            pltpu.VMEM((C * N,), jnp.float32),
            pltpu.VMEM((C * N,), jnp.float32),
            pltpu.VMEM((K, ppw), jnp.int32),
            pltpu.VMEM((K, ppw), jnp.int32),
            pltpu.VMEM((K * C, ppw), jnp.float32),
            pltpu.SemaphoreType.DMA((2,)),
            pltpu.SemaphoreType.DMA((2,)),
            pltpu.SemaphoreType.DMA((2,)),
        ],
    )
    def gather_kernel(f_hbm, i_hbm, o_hbm, tab0, tab1, idx0, idx1,
                      out0, tsem, isem, osem):
        tab_v = [tab0, tab1]
        idx_v = [idx0, idx1]
        wid = lax.axis_index("s") * _NC + lax.axis_index("c")
        base = wid * ppw

        def start_in(b, s):
            t = pltpu.async_copy(f_hbm.at[b], tab_v[s], tsem.at[s])
            i = pltpu.async_copy(i_hbm.at[b, :, pl.ds(base, ppw)],
                                 idx_v[s], isem.at[s])
            return t, i

        in_cp = {0: start_in(0, 0)}
        out_cp = {}
        for b in range(B):
            s = b % 2
            if b + 1 < B:
                in_cp[b + 1] = start_in(b + 1, 1 - s)
            t, i = in_cp.pop(b)
            t.wait()
            i.wait()
            if b >= 1:
                out_cp.pop(b - 1).wait()

            @pl.loop(0, ppw // 16)
            def _(g):
                for k in range(K):
                    knn = idx_v[s][k, pl.ds(g * 16, 16)]
                    for c in range(C):
                        addr = knn + (c * N) if c else knn
                        vals = plsc.load_gather(tab_v[s], [addr])
                        out0[k * C + c, pl.ds(g * 16, 16)] = vals

            out_cp[b] = pltpu.async_copy(
                out0, o_hbm.at[b, :, pl.ds(base, ppw)], osem.at[b % 2])
        out_cp.pop(B - 1).wait()

    return gather_kernel(fT, idxT)


def _dense_body(g_ref, f3_ref, wcat_ref, b1_ref, w2b_ref, b2_ref,
                w3_ref, b3_ref, w4_ref, b4_ref, w5_ref, b5_ref, out_ref):
    g = g_ref[0]
    f3 = f3_ref[0]
    xin = jnp.concatenate([g, f3], axis=0)
    h = jnp.dot(wcat_ref[...], xin.astype(jnp.bfloat16),
                preferred_element_type=jnp.float32)
    h = jnp.maximum(h + b1_ref[...], 0.0)
    h = jnp.dot(w2b_ref[...], h.astype(jnp.bfloat16),
                preferred_element_type=jnp.float32)
    h = jnp.maximum(h + b2_ref[...], 0.0)
    m = jnp.maximum(h[:256], h[256:])
    m = jnp.maximum(m[:128], m[128:])
    m = jnp.maximum(m[:64], m[64:])
    xmax = jnp.maximum(m[:32], m[32:])
    t = jnp.dot(w3_ref[...], f3, preferred_element_type=jnp.float32)
    t = jnp.maximum(t + b3_ref[...], 0.0)
    gsk = jnp.dot(w4_ref[...], t, preferred_element_type=jnp.float32)
    gsk = jnp.maximum(gsk + b4_ref[...], 0.0)
    s = xmax + gsk
    o8 = (jnp.dot(w5_ref[...], s, preferred_element_type=jnp.float32)
          + b5_ref[...])
    out_ref[0] = o8[:3]


def _blkdiag(w, r):
    a, b = w.shape
    out = jnp.zeros((r * a, r * b), w.dtype)
    for i in range(r):
        out = out.at[i * a:(i + 1) * a, i * b:(i + 1) * b].set(w)
    return out


def kernel(f, knn_idx, W1, b1, W2, b2, W3, b3, W4, b4, W5, b5):
    B, N, C = f.shape
    K = knn_idx.shape[2]

    fT = jnp.swapaxes(f, 1, 2)
    idxT = jnp.swapaxes(knn_idx, 1, 2)
    g3p = _sc_gather(fT.reshape(B, C * N), idxT)

    W1a = W1[:C] + W1[C:]
    W1b = W1[C:]
    wcat = jnp.concatenate([_blkdiag(W1a, K),
                            jnp.tile(-W1b, (1, K))],
                           axis=0).T.astype(jnp.bfloat16)
    b1r = jnp.tile(b1, K)[:, None]
    w2b = _blkdiag(W2, K).T.astype(jnp.bfloat16)
    b2r = jnp.tile(b2, K)[:, None]
    w3t = W3.T
    w4t = W4.T
    w5t = jnp.pad(W5.T, ((0, 5), (0, 0)))
    b5p = jnp.pad(b5, (0, 5))[:, None]

    grid = (B, N // _TP)
    out = pl.pallas_call(
        _dense_body,
        grid=grid,
        in_specs=[
            pl.BlockSpec((1, K * C, _TP), lambda b, i: (b, 0, i)),
            pl.BlockSpec((1, C, _TP), lambda b, i: (b, 0, i)),
            pl.BlockSpec((K * 32, K * C + 3), lambda b, i: (0, 0)),
            pl.BlockSpec((K * 32, 1), lambda b, i: (0, 0)),
            pl.BlockSpec((K * 32, K * 32), lambda b, i: (0, 0)),
            pl.BlockSpec((K * 32, 1), lambda b, i: (0, 0)),
            pl.BlockSpec((32, 3), lambda b, i: (0, 0)),
            pl.BlockSpec((32, 1), lambda b, i: (0, 0)),
            pl.BlockSpec((32, 32), lambda b, i: (0, 0)),
            pl.BlockSpec((32, 1), lambda b, i: (0, 0)),
            pl.BlockSpec((8, 32), lambda b, i: (0, 0)),
            pl.BlockSpec((8, 1), lambda b, i: (0, 0)),
        ],
        out_specs=pl.BlockSpec((1, 3, _TP), lambda b, i: (b, 0, i)),
        out_shape=jax.ShapeDtypeStruct((B, 3, N), jnp.float32),
    )(g3p, fT, wcat, b1r, w2b, b2r, w3t, b3[:, None], w4t, b4[:, None],
      w5t, b5p)
    return jnp.swapaxes(out, 1, 2)


# speedup vs baseline: 1.5801x; 1.0187x over previous
import dataclasses

import jax
import jax.numpy as jnp
from jax import lax
from jax.experimental import pallas as pl
from jax.experimental.pallas import tpu as pltpu
from jax.experimental.pallas import tpu_sc as plsc

_TP = 8192
_NC = 2
_NS = 16


def _prep_body(f_ref, i_ref, ft_ref, it_ref):
    fr = f_ref[0]
    z = jnp.zeros((fr.shape[0], 5), jnp.float32)
    ft_ref[0] = jnp.concatenate([fr, z], axis=1).T[:3]
    it_ref[0] = i_ref[0].T


def _prep(f, knn_idx):
    B, N, C = f.shape
    K = knn_idx.shape[2]
    return pl.pallas_call(
        _prep_body,
        grid=(B,),
        in_specs=[
            pl.BlockSpec((1, N, C), lambda b: (b, 0, 0)),
            pl.BlockSpec((1, N, K), lambda b: (b, 0, 0)),
        ],
        out_specs=[
            pl.BlockSpec((1, C, N), lambda b: (b, 0, 0)),
            pl.BlockSpec((1, K, N), lambda b: (b, 0, 0)),
        ],
        out_shape=[
            jax.ShapeDtypeStruct((B, C, N), jnp.float32),
            jax.ShapeDtypeStruct((B, K, N), jnp.int32),
        ],
    )(f, knn_idx)


def _sc_gather(fT, idxT):
    B = fT.shape[0]
    K, N = idxT.shape[1], idxT.shape[2]
    C = fT.shape[1] // N
    nw = _NC * _NS
    ppw = N // nw
    mesh = plsc.VectorSubcoreMesh(core_axis_name="c", subcore_axis_name="s")
    cp = pltpu.CompilerParams()
    if "needs_layout_passes" in pltpu.CompilerParams.__dataclass_fields__:
        cp = dataclasses.replace(cp, needs_layout_passes=False)

    @pl.kernel(
        out_type=jax.ShapeDtypeStruct((B, K * C, N), jnp.float32),
        mesh=mesh,
        compiler_params=cp,
        scratch_types=[
            pltpu.VMEM((C * N,), jnp.float32),
            pltpu.VMEM((C * N,), jnp.float32),
            pltpu.VMEM((K, ppw), jnp.int32),
            pltpu.VMEM((K, ppw), jnp.int32),
            pltpu.VMEM((K * C, ppw), jnp.float32),
            pltpu.SemaphoreType.DMA((2,)),
            pltpu.SemaphoreType.DMA((2,)),
            pltpu.SemaphoreType.DMA((2,)),
        ],
    )
    def gather_kernel(f_hbm, i_hbm, o_hbm, tab0, tab1, idx0, idx1,
                      out0, tsem, isem, osem):
        tab_v = [tab0, tab1]
        idx_v = [idx0, idx1]
        wid = lax.axis_index("s") * _NC + lax.axis_index("c")
        base = wid * ppw

        def start_in(b, s):
            t = pltpu.async_copy(f_hbm.at[b], tab_v[s], tsem.at[s])
            i = pltpu.async_copy(i_hbm.at[b, :, pl.ds(base, ppw)],
                                 idx_v[s], isem.at[s])
            return t, i

        in_cp = {0: start_in(0, 0)}
        out_cp = {}
        for b in range(B):
            s = b % 2
            if b + 1 < B:
                in_cp[b + 1] = start_in(b + 1, 1 - s)
            t, i = in_cp.pop(b)
            t.wait()
            i.wait()
            if b >= 1:
                out_cp.pop(b - 1).wait()

            @pl.loop(0, ppw // 16)
            def _(g):
                for k in range(K):
                    knn = idx_v[s][k, pl.ds(g * 16, 16)]
                    for c in range(C):
                        addr = knn + (c * N) if c else knn
                        vals = plsc.load_gather(tab_v[s], [addr])
                        out0[k * C + c, pl.ds(g * 16, 16)] = vals

            out_cp[b] = pltpu.async_copy(
                out0, o_hbm.at[b, :, pl.ds(base, ppw)], osem.at[b % 2])
        out_cp.pop(B - 1).wait()

    return gather_kernel(fT, idxT)


def _dense_body(g_ref, f3_ref, wcat_ref, b1_ref, w2b_ref, b2_ref,
                w3_ref, b3_ref, w4_ref, b4_ref, w5_ref, b5_ref, out_ref):
    g = g_ref[0]
    f3 = f3_ref[0]
    xin = jnp.concatenate([g, f3], axis=0)
    h = jnp.dot(wcat_ref[...], xin.astype(jnp.bfloat16),
                preferred_element_type=jnp.float32)
    h = jnp.maximum(h + b1_ref[...], 0.0)
    h = jnp.dot(w2b_ref[...], h.astype(jnp.bfloat16),
                preferred_element_type=jnp.float32)
    h = jnp.maximum(h + b2_ref[...], 0.0)
    m = jnp.maximum(h[:256], h[256:])
    m = jnp.maximum(m[:128], m[128:])
    m = jnp.maximum(m[:64], m[64:])
    xmax = jnp.maximum(m[:32], m[32:])
    t = jnp.dot(w3_ref[...], f3, preferred_element_type=jnp.float32)
    t = jnp.maximum(t + b3_ref[...], 0.0)
    gsk = jnp.dot(w4_ref[...], t, preferred_element_type=jnp.float32)
    gsk = jnp.maximum(gsk + b4_ref[...], 0.0)
    s = xmax + gsk
    o8 = (jnp.dot(w5_ref[...], s, preferred_element_type=jnp.float32)
          + b5_ref[...])
    out_ref[0] = o8[:3]


def _blkdiag(w, r):
    a, b = w.shape
    out = jnp.zeros((r * a, r * b), w.dtype)
    for i in range(r):
        out = out.at[i * a:(i + 1) * a, i * b:(i + 1) * b].set(w)
    return out


def kernel(f, knn_idx, W1, b1, W2, b2, W3, b3, W4, b4, W5, b5):
    B, N, C = f.shape
    K = knn_idx.shape[2]

    fT = jnp.swapaxes(f, 1, 2)
    idxT = jnp.swapaxes(knn_idx, 1, 2)
    g3p = _sc_gather(fT.reshape(B, C * N), idxT)

    W1a = W1[:C] + W1[C:]
    W1b = W1[C:]
    wcat = jnp.concatenate([_blkdiag(W1a, K),
                            jnp.tile(-W1b, (1, K))],
                           axis=0).T.astype(jnp.bfloat16)
    b1r = jnp.tile(b1, K)[:, None]
    w2b = _blkdiag(W2, K).T.astype(jnp.bfloat16)
    b2r = jnp.tile(b2, K)[:, None]
    w3t = W3.T
    w4t = W4.T
    w5t = jnp.pad(W5.T, ((0, 5), (0, 0)))
    b5p = jnp.pad(b5, (0, 5))[:, None]

    grid = (B, N // _TP)
    out = pl.pallas_call(
        _dense_body,
        grid=grid,
        in_specs=[
            pl.BlockSpec((1, K * C, _TP), lambda b, i: (b, 0, i)),
            pl.BlockSpec((1, C, _TP), lambda b, i: (b, 0, i)),
            pl.BlockSpec((K * 32, K * C + 3), lambda b, i: (0, 0)),
            pl.BlockSpec((K * 32, 1), lambda b, i: (0, 0)),
            pl.BlockSpec((K * 32, K * 32), lambda b, i: (0, 0)),
            pl.BlockSpec((K * 32, 1), lambda b, i: (0, 0)),
            pl.BlockSpec((32, 3), lambda b, i: (0, 0)),
            pl.BlockSpec((32, 1), lambda b, i: (0, 0)),
            pl.BlockSpec((32, 32), lambda b, i: (0, 0)),
            pl.BlockSpec((32, 1), lambda b, i: (0, 0)),
            pl.BlockSpec((8, 32), lambda b, i: (0, 0)),
            pl.BlockSpec((8, 1), lambda b, i: (0, 0)),
        ],
        out_specs=pl.BlockSpec((1, 3, _TP), lambda b, i: (b, 0, i)),
        out_shape=jax.ShapeDtypeStruct((B, 3, N), jnp.float32),
    )(g3p, fT, wcat, b1r, w2b, b2r, w3t, b3[:, None], w4t, b4[:, None],
      w5t, b5p)
    return jnp.swapaxes(out, 1, 2)
